# Initial kernel scaffold; baseline (speedup 1.0000x reference)
import jax
import jax.numpy as jnp
from jax.experimental import pallas as pl


def kernel(x, view2, edge_index, D_inv, W1, b1, W2, b2, W3, b3, W4, b4, W5, b5, W6, b6):
    raise NotImplementedError("write your pallas kernel here")


# speedup vs baseline: 4.5184x; 4.5184x over previous
import functools

import jax
import jax.numpy as jnp
from jax import lax
from jax.experimental import pallas as pl
from jax.experimental.pallas import tpu as pltpu
from jax.experimental.pallas import tpu_sc as plsc

N = 10000
E = 320000
D_IN = 128
H = 256
HQ = 128
NQ = 2
NC = 2
NT = 16

EPT = E // NT
KB = 128
IC = 16
NCH = 10
EPT_P = NCH * IC * KB
PAD_T = EPT_P - EPT
TRASH = N
ACC_ROWS = N + 8

EPT_A = E // (NC * NT)
KA = 100
NBLK_A = EPT_A // KA

ROWS_PER_TILE = 1000
FLUSH_TILES = N // ROWS_PER_TILE

_MESH = plsc.VectorSubcoreMesh(
    core_axis_name="c", subcore_axis_name="s", num_cores=NC, num_subcores=NT
)


def _deg_body(edst4, ones_hbm, zeros_hbm, deg2, dst_v, ones_v, acc_sh, sem):
    c = lax.axis_index("c")
    s = lax.axis_index("s")
    pltpu.sync_copy(edst4.at[c, s], dst_v)
    pltpu.sync_copy(ones_hbm, ones_v)

    @pl.when(s == 0)
    def _():
        pltpu.sync_copy(zeros_hbm, acc_sh)

    plsc.subcore_barrier()

    def step(j, carry):
        pltpu.sync_copy(ones_v, acc_sh.at[dst_v.at[j]], add=True)
        return carry

    lax.fori_loop(0, NBLK_A, step, 0)
    plsc.subcore_barrier()

    @pl.when(s == 0)
    def _():
        pltpu.sync_copy(acc_sh, deg2.at[c])

    del sem


_deg_kernel = pl.kernel(
    _deg_body,
    out_type=jax.ShapeDtypeStruct((NC, N), jnp.float32),
    mesh=_MESH,
    scratch_types=[
        pltpu.VMEM((NBLK_A, KA), jnp.int32),
        pltpu.VMEM((KA,), jnp.float32),
        pltpu.VMEM_SHARED((N,), jnp.float32),
        pltpu.SemaphoreType.DMA,
    ],
)


def _scatter_body(s4, esrc4, edst4, zeros_hbm, acc4,
                  src_v, dst_v, rows_a, rows_b, acc_sh, sem_a, sem_b):
    c = lax.axis_index("c")
    s = lax.axis_index("s")

    for q in range(NQ):
        row0 = s * ROWS_PER_TILE

        @pl.when(s < FLUSH_TILES)
        def _():
            pltpu.sync_copy(zeros_hbm,
                            acc_sh.at[pl.ds(row0, ROWS_PER_TILE)])

        plsc.subcore_barrier()

        table = s4.at[c, q]

        def chunk(ch, carry):
            pltpu.sync_copy(esrc4.at[s, ch], src_v)
            pltpu.sync_copy(edst4.at[s, ch], dst_v)
            pltpu.async_copy(table.at[src_v.at[0]], rows_a, sem_a)

            def step(i, carry2):
                b0 = 2 * i
                b1 = b0 + 1
                pltpu.make_async_copy(table.at[src_v.at[b0]],
                                      rows_a, sem_a).wait()
                pltpu.async_copy(table.at[src_v.at[b1]], rows_b, sem_b)
                pltpu.sync_copy(rows_a, acc_sh.at[dst_v.at[b0]], add=True)
                pltpu.make_async_copy(table.at[src_v.at[b1]],
                                      rows_b, sem_b).wait()

                @pl.when(i < IC // 2 - 1)
                def _():
                    pltpu.async_copy(table.at[src_v.at[b0 + 2]], rows_a, sem_a)

                pltpu.sync_copy(rows_b, acc_sh.at[dst_v.at[b1]], add=True)
                return carry2

            lax.fori_loop(0, IC // 2, step, 0)
            return carry

        lax.fori_loop(0, NCH, chunk, 0)
        plsc.subcore_barrier()

        @pl.when(s < FLUSH_TILES)
        def _():
            pltpu.sync_copy(acc_sh.at[pl.ds(row0, ROWS_PER_TILE)],
                            acc4.at[c, q, pl.ds(row0, ROWS_PER_TILE)])

        plsc.subcore_barrier()


_scatter_kernel = pl.kernel(
    _scatter_body,
    out_type=jax.ShapeDtypeStruct((NC, NQ, N, HQ), jnp.float32),
    mesh=_MESH,
    scratch_types=[
        pltpu.VMEM((IC, KB), jnp.int32),
        pltpu.VMEM((IC, KB), jnp.int32),
        pltpu.VMEM((KB, HQ), jnp.float32),
        pltpu.VMEM((KB, HQ), jnp.float32),
        pltpu.VMEM_SHARED((ACC_ROWS, HQ), jnp.float32),
        pltpu.SemaphoreType.DMA,
        pltpu.SemaphoreType.DMA,
    ],
)


_R = 2000
_GRID = N // _R


def _split_to_s4(s4_ref, s1, s2):
    s4_ref[0, 0] = s1[:, :HQ]
    s4_ref[0, 1] = s1[:, HQ:]
    s4_ref[1, 0] = s2[:, :HQ]
    s4_ref[1, 1] = s2[:, HQ:]


def _tc1_body(x_ref, v2_ref, dega_ref, degb_ref,
              w1_ref, b1_ref, w4_ref, b4_ref, s4_ref, dis_ref):
    deg = dega_ref[...] + degb_ref[...]
    dis = jnp.where(deg > 0, lax.rsqrt(jnp.maximum(deg, 1.0)), 0.0)
    dis_ref[...] = dis
    s1 = (jnp.dot(x_ref[...], w1_ref[...],
                  preferred_element_type=jnp.float32) + b1_ref[...]) * dis
    s2 = (jnp.dot(v2_ref[...], w4_ref[...],
                  preferred_element_type=jnp.float32) + b4_ref[...]) * dis
    _split_to_s4(s4_ref, s1, s2)


_tc1 = pl.pallas_call(
    _tc1_body,
    grid=(_GRID,),
    in_specs=[
        pl.BlockSpec((_R, D_IN), lambda i: (i, 0)),
        pl.BlockSpec((_R, D_IN), lambda i: (i, 0)),
        pl.BlockSpec((_R, 1), lambda i: (i, 0)),
        pl.BlockSpec((_R, 1), lambda i: (i, 0)),
        pl.BlockSpec((D_IN, H), lambda i: (0, 0)),
        pl.BlockSpec((1, H), lambda i: (0, 0)),
        pl.BlockSpec((D_IN, H), lambda i: (0, 0)),
        pl.BlockSpec((1, H), lambda i: (0, 0)),
    ],
    out_specs=[
        pl.BlockSpec((NC, NQ, _R, HQ), lambda i: (0, 0, i, 0)),
        pl.BlockSpec((_R, 1), lambda i: (i, 0)),
    ],
    out_shape=[
        jax.ShapeDtypeStruct((NC, NQ, N, HQ), jnp.float32),
        jax.ShapeDtypeStruct((N, 1), jnp.float32),
    ],
)


def _tc_mid_body(acc_ref, dis_ref, wa_ref, ba_ref, wb_ref, bb_ref,
                 y_ref, s4_ref):
    dis = dis_ref[...]
    a1 = jnp.concatenate([acc_ref[0, 0], acc_ref[0, 1]], axis=1)
    a2 = jnp.concatenate([acc_ref[1, 0], acc_ref[1, 1]], axis=1)
    y1 = jnp.maximum(a1 * dis, 0.0)
    y2 = jnp.maximum(a2 * dis, 0.0)
    y_ref[0] = y1
    y_ref[1] = y2
    s1 = (jnp.dot(y1, wa_ref[...],
                  preferred_element_type=jnp.float32) + ba_ref[...]) * dis
    s2 = (jnp.dot(y2, wb_ref[...],
                  preferred_element_type=jnp.float32) + bb_ref[...]) * dis
    _split_to_s4(s4_ref, s1, s2)


_tc_mid = pl.pallas_call(
    _tc_mid_body,
    grid=(_GRID,),
    in_specs=[
        pl.BlockSpec((NC, NQ, _R, HQ), lambda i: (0, 0, i, 0)),
        pl.BlockSpec((_R, 1), lambda i: (i, 0)),
        pl.BlockSpec((H, H), lambda i: (0, 0)),
        pl.BlockSpec((1, H), lambda i: (0, 0)),
        pl.BlockSpec((H, H), lambda i: (0, 0)),
        pl.BlockSpec((1, H), lambda i: (0, 0)),
    ],
    out_specs=[
        pl.BlockSpec((NC, _R, H), lambda i: (0, i, 0)),
        pl.BlockSpec((NC, NQ, _R, HQ), lambda i: (0, 0, i, 0)),
    ],
    out_shape=[
        jax.ShapeDtypeStruct((NC, N, H), jnp.float32),
        jax.ShapeDtypeStruct((NC, NQ, N, HQ), jnp.float32),
    ],
)


def _tc_fin_body(acc_ref, dis_ref, y_ref):
    dis = dis_ref[...]
    a1 = jnp.concatenate([acc_ref[0, 0], acc_ref[0, 1]], axis=1)
    a2 = jnp.concatenate([acc_ref[1, 0], acc_ref[1, 1]], axis=1)
    y_ref[0] = jnp.maximum(a1 * dis, 0.0)
    y_ref[1] = jnp.maximum(a2 * dis, 0.0)


_tc_fin = pl.pallas_call(
    _tc_fin_body,
    grid=(_GRID,),
    in_specs=[
        pl.BlockSpec((NC, NQ, _R, HQ), lambda i: (0, 0, i, 0)),
        pl.BlockSpec((_R, 1), lambda i: (i, 0)),
    ],
    out_specs=pl.BlockSpec((NC, _R, H), lambda i: (0, i, 0)),
    out_shape=jax.ShapeDtypeStruct((NC, N, H), jnp.float32),
)


def kernel(x, view2, edge_index, D_inv,
           W1, b1, W2, b2, W3, b3, W4, b4, W5, b5, W6, b6):
    del D_inv
    esrc = edge_index[0]
    edst = edge_index[1]
    esrc4 = jnp.concatenate(
        [esrc.reshape(NT, EPT),
         jnp.zeros((NT, PAD_T), jnp.int32)], axis=1,
    ).reshape(NT, NCH, IC, KB)
    edst4s = jnp.concatenate(
        [edst.reshape(NT, EPT),
         jnp.full((NT, PAD_T), TRASH, jnp.int32)], axis=1,
    ).reshape(NT, NCH, IC, KB)
    edst4a = edst.reshape(NC, NT, NBLK_A, KA)
    ones_a = jnp.ones((KA,), jnp.float32)
    zeros_n = jnp.zeros((N,), jnp.float32)
    zeros_nq = jnp.zeros((ROWS_PER_TILE, HQ), jnp.float32)
    b1r = b1.reshape(1, H)
    b2r = b2.reshape(1, H)
    b3r = b3.reshape(1, H)
    b4r = b4.reshape(1, H)
    b5r = b5.reshape(1, H)
    b6r = b6.reshape(1, H)

    deg2 = _deg_kernel(edst4a, ones_a, zeros_n)
    dega = deg2[0].reshape(N, 1)
    degb = deg2[1].reshape(N, 1)

    s4_1, dis = _tc1(x, view2, dega, degb, W1, b1r, W4, b4r)
    acc1 = _scatter_kernel(s4_1, esrc4, edst4s, zeros_nq)
    y1, s4_2 = _tc_mid(acc1, dis, W2, b2r, W5, b5r)
    acc2 = _scatter_kernel(s4_2, esrc4, edst4s, zeros_nq)
    y2, s4_3 = _tc_mid(acc2, dis, W3, b3r, W6, b6r)
    acc3 = _scatter_kernel(s4_3, esrc4, edst4s, zeros_nq)
    y3 = _tc_fin(acc3, dis)

    q = jnp.concatenate([y1[0], y2[0], y3[0]], axis=1)
    p = jnp.concatenate([y1[1], y2[1], y3[1]], axis=1)
    return (q, p)


# speedup vs baseline: 5.0344x; 1.1142x over previous
import functools

import jax
import jax.numpy as jnp
from jax import lax
from jax.experimental import pallas as pl
from jax.experimental.pallas import tpu as pltpu
from jax.experimental.pallas import tpu_sc as plsc

N = 10000
E = 320000
D_IN = 128
H = 256
HQ = 128
NQ = 2
NC = 2
NT = 16

EPT = E // NT
KB = 64
IC = 32
NCH = 10
NBUF = 4
EPT_P = NCH * IC * KB
PAD_T = EPT_P - EPT
TRASH = N
ACC_ROWS = N + 8

EPT_A = E // (NC * NT)
KA = 100
NBLK_A = EPT_A // KA

ROWS_PER_TILE = 1000
FLUSH_TILES = N // ROWS_PER_TILE

_MESH = plsc.VectorSubcoreMesh(
    core_axis_name="c", subcore_axis_name="s", num_cores=NC, num_subcores=NT
)


def _deg_body(edst4, ones_hbm, zeros_hbm, deg2, dst_v, ones_v, acc_sh, sem):
    c = lax.axis_index("c")
    s = lax.axis_index("s")
    pltpu.sync_copy(edst4.at[c, s], dst_v)
    pltpu.sync_copy(ones_hbm, ones_v)

    @pl.when(s == 0)
    def _():
        pltpu.sync_copy(zeros_hbm, acc_sh)

    plsc.subcore_barrier()

    def step(j, carry):
        pltpu.sync_copy(ones_v, acc_sh.at[dst_v.at[j]], add=True)
        return carry

    lax.fori_loop(0, NBLK_A, step, 0)
    plsc.subcore_barrier()

    @pl.when(s == 0)
    def _():
        pltpu.sync_copy(acc_sh, deg2.at[c])

    del sem


_deg_kernel = pl.kernel(
    _deg_body,
    out_type=jax.ShapeDtypeStruct((NC, N), jnp.float32),
    mesh=_MESH,
    scratch_types=[
        pltpu.VMEM((NBLK_A, KA), jnp.int32),
        pltpu.VMEM((KA,), jnp.float32),
        pltpu.VMEM_SHARED((N,), jnp.float32),
        pltpu.SemaphoreType.DMA,
    ],
)


def _scatter_body(s4, esrc4, edst4, zeros_hbm, acc4,
                  src_v, dst_v, rows_a, rows_b, rows_c, rows_d,
                  acc_sh, sem_a, sem_b, sem_c, sem_d):
    c = lax.axis_index("c")
    s = lax.axis_index("s")
    rows = (rows_a, rows_b, rows_c, rows_d)
    sems = (sem_a, sem_b, sem_c, sem_d)

    for q in range(NQ):
        row0 = s * ROWS_PER_TILE

        @pl.when(s < FLUSH_TILES)
        def _():
            pltpu.sync_copy(zeros_hbm,
                            acc_sh.at[pl.ds(row0, ROWS_PER_TILE)])

        plsc.subcore_barrier()

        table = s4.at[c, q]

        def chunk(ch, carry):
            pltpu.sync_copy(esrc4.at[s, ch], src_v)
            pltpu.sync_copy(edst4.at[s, ch], dst_v)
            for k in range(NBUF):
                pltpu.async_copy(table.at[src_v.at[k]], rows[k], sems[k])

            def step(i, carry2):
                for k in range(NBUF):
                    b = NBUF * i + k
                    pltpu.make_async_copy(table.at[src_v.at[b]],
                                          rows[k], sems[k]).wait()
                    pltpu.sync_copy(rows[k], acc_sh.at[dst_v.at[b]], add=True)

                    @pl.when(b + NBUF < IC)
                    def _():
                        pltpu.async_copy(table.at[src_v.at[b + NBUF]],
                                         rows[k], sems[k])
                return carry2

            lax.fori_loop(0, IC // NBUF, step, 0)
            return carry

        lax.fori_loop(0, NCH, chunk, 0)
        plsc.subcore_barrier()

        @pl.when(s < FLUSH_TILES)
        def _():
            pltpu.sync_copy(acc_sh.at[pl.ds(row0, ROWS_PER_TILE)],
                            acc4.at[c, q, pl.ds(row0, ROWS_PER_TILE)])

        plsc.subcore_barrier()


_scatter_kernel = pl.kernel(
    _scatter_body,
    out_type=jax.ShapeDtypeStruct((NC, NQ, N, HQ), jnp.float32),
    mesh=_MESH,
    scratch_types=[
        pltpu.VMEM((IC, KB), jnp.int32),
        pltpu.VMEM((IC, KB), jnp.int32),
        pltpu.VMEM((KB, HQ), jnp.float32),
        pltpu.VMEM((KB, HQ), jnp.float32),
        pltpu.VMEM((KB, HQ), jnp.float32),
        pltpu.VMEM((KB, HQ), jnp.float32),
        pltpu.VMEM_SHARED((ACC_ROWS, HQ), jnp.float32),
        pltpu.SemaphoreType.DMA,
        pltpu.SemaphoreType.DMA,
        pltpu.SemaphoreType.DMA,
        pltpu.SemaphoreType.DMA,
    ],
)


_R = 2000
_GRID = N // _R


def _split_to_s4(s4_ref, s1, s2):
    s4_ref[0, 0] = s1[:, :HQ]
    s4_ref[0, 1] = s1[:, HQ:]
    s4_ref[1, 0] = s2[:, :HQ]
    s4_ref[1, 1] = s2[:, HQ:]


def _tc1_body(x_ref, v2_ref, dega_ref, degb_ref,
              w1_ref, b1_ref, w4_ref, b4_ref, s4_ref, dis_ref):
    deg = dega_ref[...] + degb_ref[...]
    dis = jnp.where(deg > 0, lax.rsqrt(jnp.maximum(deg, 1.0)), 0.0)
    dis_ref[...] = dis
    s1 = (jnp.dot(x_ref[...], w1_ref[...],
                  preferred_element_type=jnp.float32) + b1_ref[...]) * dis
    s2 = (jnp.dot(v2_ref[...], w4_ref[...],
                  preferred_element_type=jnp.float32) + b4_ref[...]) * dis
    _split_to_s4(s4_ref, s1, s2)


_tc1 = pl.pallas_call(
    _tc1_body,
    grid=(_GRID,),
    in_specs=[
        pl.BlockSpec((_R, D_IN), lambda i: (i, 0)),
        pl.BlockSpec((_R, D_IN), lambda i: (i, 0)),
        pl.BlockSpec((_R, 1), lambda i: (i, 0)),
        pl.BlockSpec((_R, 1), lambda i: (i, 0)),
        pl.BlockSpec((D_IN, H), lambda i: (0, 0)),
        pl.BlockSpec((1, H), lambda i: (0, 0)),
        pl.BlockSpec((D_IN, H), lambda i: (0, 0)),
        pl.BlockSpec((1, H), lambda i: (0, 0)),
    ],
    out_specs=[
        pl.BlockSpec((NC, NQ, _R, HQ), lambda i: (0, 0, i, 0)),
        pl.BlockSpec((_R, 1), lambda i: (i, 0)),
    ],
    out_shape=[
        jax.ShapeDtypeStruct((NC, NQ, N, HQ), jnp.float32),
        jax.ShapeDtypeStruct((N, 1), jnp.float32),
    ],
)


def _tc_mid_body(acc_ref, dis_ref, wa_ref, ba_ref, wb_ref, bb_ref,
                 y_ref, s4_ref):
    dis = dis_ref[...]
    a1 = jnp.concatenate([acc_ref[0, 0], acc_ref[0, 1]], axis=1)
    a2 = jnp.concatenate([acc_ref[1, 0], acc_ref[1, 1]], axis=1)
    y1 = jnp.maximum(a1 * dis, 0.0)
    y2 = jnp.maximum(a2 * dis, 0.0)
    y_ref[0] = y1
    y_ref[1] = y2
    s1 = (jnp.dot(y1, wa_ref[...],
                  preferred_element_type=jnp.float32) + ba_ref[...]) * dis
    s2 = (jnp.dot(y2, wb_ref[...],
                  preferred_element_type=jnp.float32) + bb_ref[...]) * dis
    _split_to_s4(s4_ref, s1, s2)


_tc_mid = pl.pallas_call(
    _tc_mid_body,
    grid=(_GRID,),
    in_specs=[
        pl.BlockSpec((NC, NQ, _R, HQ), lambda i: (0, 0, i, 0)),
        pl.BlockSpec((_R, 1), lambda i: (i, 0)),
        pl.BlockSpec((H, H), lambda i: (0, 0)),
        pl.BlockSpec((1, H), lambda i: (0, 0)),
        pl.BlockSpec((H, H), lambda i: (0, 0)),
        pl.BlockSpec((1, H), lambda i: (0, 0)),
    ],
    out_specs=[
        pl.BlockSpec((NC, _R, H), lambda i: (0, i, 0)),
        pl.BlockSpec((NC, NQ, _R, HQ), lambda i: (0, 0, i, 0)),
    ],
    out_shape=[
        jax.ShapeDtypeStruct((NC, N, H), jnp.float32),
        jax.ShapeDtypeStruct((NC, NQ, N, HQ), jnp.float32),
    ],
)


def _tc_fin_body(acc_ref, dis_ref, y_ref):
    dis = dis_ref[...]
    a1 = jnp.concatenate([acc_ref[0, 0], acc_ref[0, 1]], axis=1)
    a2 = jnp.concatenate([acc_ref[1, 0], acc_ref[1, 1]], axis=1)
    y_ref[0] = jnp.maximum(a1 * dis, 0.0)
    y_ref[1] = jnp.maximum(a2 * dis, 0.0)


_tc_fin = pl.pallas_call(
    _tc_fin_body,
    grid=(_GRID,),
    in_specs=[
        pl.BlockSpec((NC, NQ, _R, HQ), lambda i: (0, 0, i, 0)),
        pl.BlockSpec((_R, 1), lambda i: (i, 0)),
    ],
    out_specs=pl.BlockSpec((NC, _R, H), lambda i: (0, i, 0)),
    out_shape=jax.ShapeDtypeStruct((NC, N, H), jnp.float32),
)


def kernel(x, view2, edge_index, D_inv,
           W1, b1, W2, b2, W3, b3, W4, b4, W5, b5, W6, b6):
    del D_inv
    esrc = edge_index[0]
    edst = edge_index[1]
    esrc4 = jnp.concatenate(
        [esrc.reshape(NT, EPT),
         jnp.zeros((NT, PAD_T), jnp.int32)], axis=1,
    ).reshape(NT, NCH, IC, KB)
    edst4s = jnp.concatenate(
        [edst.reshape(NT, EPT),
         jnp.full((NT, PAD_T), TRASH, jnp.int32)], axis=1,
    ).reshape(NT, NCH, IC, KB)
    edst4a = edst.reshape(NC, NT, NBLK_A, KA)
    ones_a = jnp.ones((KA,), jnp.float32)
    zeros_n = jnp.zeros((N,), jnp.float32)
    zeros_nq = jnp.zeros((ROWS_PER_TILE, HQ), jnp.float32)
    b1r = b1.reshape(1, H)
    b2r = b2.reshape(1, H)
    b3r = b3.reshape(1, H)
    b4r = b4.reshape(1, H)
    b5r = b5.reshape(1, H)
    b6r = b6.reshape(1, H)

    deg2 = _deg_kernel(edst4a, ones_a, zeros_n)
    dega = deg2[0].reshape(N, 1)
    degb = deg2[1].reshape(N, 1)

    s4_1, dis = _tc1(x, view2, dega, degb, W1, b1r, W4, b4r)
    acc1 = _scatter_kernel(s4_1, esrc4, edst4s, zeros_nq)
    y1, s4_2 = _tc_mid(acc1, dis, W2, b2r, W5, b5r)
    acc2 = _scatter_kernel(s4_2, esrc4, edst4s, zeros_nq)
    y2, s4_3 = _tc_mid(acc2, dis, W3, b3r, W6, b6r)
    acc3 = _scatter_kernel(s4_3, esrc4, edst4s, zeros_nq)
    y3 = _tc_fin(acc3, dis)

    q = jnp.concatenate([y1[0], y2[0], y3[0]], axis=1)
    p = jnp.concatenate([y1[1], y2[1], y3[1]], axis=1)
    return (q, p)


# speedup vs baseline: 5.8607x; 1.1641x over previous
import functools

import jax
import jax.numpy as jnp
from jax import lax
from jax.experimental import pallas as pl
from jax.experimental.pallas import tpu as pltpu
from jax.experimental.pallas import tpu_sc as plsc

N = 10000
E = 320000
D_IN = 128
H = 256
NC = 2
NT = 16

NHALF = N // 2
TRASH = NHALF
ACC_R = 2 * (NHALF + 8)

KB = 64
IC = 20
CHUNK = IC * KB
WT_E = E // 32
LCAP_CH = 8
LCAP = LCAP_CH * CHUNK

KA = 100
NBLK_A = WT_E // KA

ROWS_PER_TILE = 2000
FLUSH_TILES = 2 * NHALF // ROWS_PER_TILE

_MESH = plsc.VectorSubcoreMesh(
    core_axis_name="c", subcore_axis_name="s", num_cores=NC, num_subcores=NT
)


def _deg_body(edst4, ones_hbm, zeros_hbm, deg2, dst_v, ones_v, acc_sh, sem):
    c = lax.axis_index("c")
    s = lax.axis_index("s")
    pltpu.sync_copy(edst4.at[c, s], dst_v)
    pltpu.sync_copy(ones_hbm, ones_v)

    @pl.when(s == 0)
    def _():
        pltpu.sync_copy(zeros_hbm, acc_sh)

    plsc.subcore_barrier()

    def step(j, carry):
        pltpu.sync_copy(ones_v, acc_sh.at[dst_v.at[j]], add=True)
        return carry

    lax.fori_loop(0, NBLK_A, step, 0)
    plsc.subcore_barrier()

    @pl.when(s == 0)
    def _():
        pltpu.sync_copy(acc_sh, deg2.at[c])

    del sem


_deg_kernel = pl.kernel(
    _deg_body,
    out_type=jax.ShapeDtypeStruct((NC, N), jnp.float32),
    mesh=_MESH,
    scratch_types=[
        pltpu.VMEM((NBLK_A, KA), jnp.int32),
        pltpu.VMEM((KA,), jnp.float32),
        pltpu.VMEM_SHARED((N,), jnp.float32),
        pltpu.SemaphoreType.DMA,
    ],
)


def _bucket_body(esrc3, edst3, srcl, dstl, cnts,
                 src_v, dst_v, ls, ld, hs, hd, cnt_v):
    c = lax.axis_index("c")
    s = lax.axis_index("s")
    w = c * NT + s
    pltpu.sync_copy(esrc3.at[c, s], src_v)
    pltpu.sync_copy(edst3.at[c, s], dst_v)

    par = lax.rem(lax.iota(jnp.int32, 16), 2)
    zl = par
    tl = 2 * TRASH + par

    def fill2(j, carry):
        ls[pl.ds(16 * j, 16)] = zl
        hs[pl.ds(16 * j, 16)] = zl
        ld[pl.ds(16 * j, 16)] = tl
        hd[pl.ds(16 * j, 16)] = tl
        return carry

    lax.fori_loop(0, 2 * LCAP // 16, fill2, 0)

    def part(i, carry):
        nlo, nhi = carry
        sv = src_v[pl.ds(16 * i, 16)]
        dv = dst_v[pl.ds(16 * i, 16)]
        mlo = dv < NHALF
        mhi = jnp.logical_not(mlo)
        ilo = plsc.cumsum(mlo.astype(jnp.int32))
        ihi = plsc.cumsum(mhi.astype(jnp.int32))
        plo = nlo + ilo - 1
        phi = nhi + ihi - 1
        sv2 = 2 * sv
        dlo2 = 2 * dv
        dhi2 = 2 * (dv - NHALF)
        plsc.store_scatter(ls, [2 * plo], sv2, mask=mlo)
        plsc.store_scatter(ls, [2 * plo + 1], sv2 + 1, mask=mlo)
        plsc.store_scatter(ld, [2 * plo], dlo2, mask=mlo)
        plsc.store_scatter(ld, [2 * plo + 1], dlo2 + 1, mask=mlo)
        plsc.store_scatter(hs, [2 * phi], sv2, mask=mhi)
        plsc.store_scatter(hs, [2 * phi + 1], sv2 + 1, mask=mhi)
        plsc.store_scatter(hd, [2 * phi], dhi2, mask=mhi)
        plsc.store_scatter(hd, [2 * phi + 1], dhi2 + 1, mask=mhi)
        cl = jnp.max(ilo)
        return (nlo + cl, nhi + (16 - cl))

    nlo, nhi = lax.fori_loop(0, WT_E // 16, part, (0, 0))

    pltpu.sync_copy(ls, srcl.at[w, 0])
    pltpu.sync_copy(ld, dstl.at[w, 0])
    pltpu.sync_copy(hs, srcl.at[w, 1])
    pltpu.sync_copy(hd, dstl.at[w, 1])
    cnt_v[...] = jnp.full((16,), 1, jnp.int32) * nlo
    pltpu.sync_copy(cnt_v, cnts.at[w])
    del nhi


_bucket_kernel = pl.kernel(
    _bucket_body,
    compiler_params=pltpu.CompilerParams(needs_layout_passes=False),
    out_type=(
        jax.ShapeDtypeStruct((32, 2, 2 * LCAP), jnp.int32),
        jax.ShapeDtypeStruct((32, 2, 2 * LCAP), jnp.int32),
        jax.ShapeDtypeStruct((32, 16), jnp.int32),
    ),
    mesh=_MESH,
    scratch_types=[
        pltpu.VMEM((WT_E,), jnp.int32),
        pltpu.VMEM((WT_E,), jnp.int32),
        pltpu.VMEM((2 * LCAP,), jnp.int32),
        pltpu.VMEM((2 * LCAP,), jnp.int32),
        pltpu.VMEM((2 * LCAP,), jnp.int32),
        pltpu.VMEM((2 * LCAP,), jnp.int32),
        pltpu.VMEM((16,), jnp.int32),
    ],
)


def _scatter_body(s_full, srcl, dstl, cnts, zeros_hbm, accf,
                  idx_s, idx_d, rows_a, rows_b, cnt_v, acc_sh, sem_a, sem_b):
    c = lax.axis_index("c")
    s = lax.axis_index("s")
    pltpu.sync_copy(cnts, cnt_v)
    table = s_full.at[c]
    row0 = s * ROWS_PER_TILE

    for h in range(2):
        @pl.when(s < FLUSH_TILES)
        def _():
            pltpu.sync_copy(zeros_hbm, acc_sh.at[pl.ds(row0, ROWS_PER_TILE)])

        plsc.subcore_barrier()

        for li in range(2):
            w = 2 * s + li
            nlo = cnt_v[w][0]
            n = nlo if h == 0 else WT_E - nlo
            nch = (n + (CHUNK - 1)) // CHUNK

            def chunk(ch, carry):
                pltpu.sync_copy(srcl.at[w, h, ch], idx_s)
                pltpu.sync_copy(dstl.at[w, h, ch], idx_d)
                pltpu.async_copy(table.at[idx_s.at[0]], rows_a, sem_a)

                def step(i, carry2):
                    b0 = 2 * i
                    b1 = b0 + 1
                    pltpu.make_async_copy(table.at[idx_s.at[b0]],
                                          rows_a, sem_a).wait()
                    pltpu.async_copy(table.at[idx_s.at[b1]], rows_b, sem_b)
                    pltpu.sync_copy(rows_a,
                                    acc_sh.at[idx_d.at[b0]], add=True)
                    pltpu.make_async_copy(table.at[idx_s.at[b1]],
                                          rows_b, sem_b).wait()

                    @pl.when(i < IC // 2 - 1)
                    def _():
                        pltpu.async_copy(table.at[idx_s.at[b0 + 2]],
                                         rows_a, sem_a)

                    pltpu.sync_copy(rows_b,
                                    acc_sh.at[idx_d.at[b1]], add=True)
                    return carry2

                lax.fori_loop(0, IC // 2, step, 0)
                return carry

            lax.fori_loop(0, nch, chunk, 0)

        plsc.subcore_barrier()

        @pl.when(s < FLUSH_TILES)
        def _():
            pltpu.sync_copy(
                acc_sh.at[pl.ds(row0, ROWS_PER_TILE)],
                accf.at[c, pl.ds(h * N + row0, ROWS_PER_TILE)])

        plsc.subcore_barrier()


_scatter_kernel = pl.kernel(
    _scatter_body,
    out_type=jax.ShapeDtypeStruct((NC, 2 * N, H // 2), jnp.float32),
    mesh=_MESH,
    scratch_types=[
        pltpu.VMEM((IC, 2 * KB), jnp.int32),
        pltpu.VMEM((IC, 2 * KB), jnp.int32),
        pltpu.VMEM((2 * KB, H // 2), jnp.float32),
        pltpu.VMEM((2 * KB, H // 2), jnp.float32),
        pltpu.VMEM((32, 16), jnp.int32),
        pltpu.VMEM_SHARED((ACC_R, H // 2), jnp.float32),
        pltpu.SemaphoreType.DMA,
        pltpu.SemaphoreType.DMA,
    ],
)


_R = 2000
_GRID = N // _R


def _tc1_body(x_ref, v2_ref, dega_ref, degb_ref,
              w1_ref, b1_ref, w4_ref, b4_ref, s_ref, dis_ref):
    deg = dega_ref[...] + degb_ref[...]
    dis = jnp.where(deg > 0, lax.rsqrt(jnp.maximum(deg, 1.0)), 0.0)
    dis_ref[...] = dis
    s_ref[0] = (jnp.dot(x_ref[...], w1_ref[...],
                        preferred_element_type=jnp.float32) + b1_ref[...]) * dis
    s_ref[1] = (jnp.dot(v2_ref[...], w4_ref[...],
                        preferred_element_type=jnp.float32) + b4_ref[...]) * dis


_tc1 = pl.pallas_call(
    _tc1_body,
    grid=(_GRID,),
    in_specs=[
        pl.BlockSpec((_R, D_IN), lambda i: (i, 0)),
        pl.BlockSpec((_R, D_IN), lambda i: (i, 0)),
        pl.BlockSpec((_R, 1), lambda i: (i, 0)),
        pl.BlockSpec((_R, 1), lambda i: (i, 0)),
        pl.BlockSpec((D_IN, H), lambda i: (0, 0)),
        pl.BlockSpec((1, H), lambda i: (0, 0)),
        pl.BlockSpec((D_IN, H), lambda i: (0, 0)),
        pl.BlockSpec((1, H), lambda i: (0, 0)),
    ],
    out_specs=[
        pl.BlockSpec((NC, _R, H), lambda i: (0, i, 0)),
        pl.BlockSpec((_R, 1), lambda i: (i, 0)),
    ],
    out_shape=[
        jax.ShapeDtypeStruct((NC, N, H), jnp.float32),
        jax.ShapeDtypeStruct((N, 1), jnp.float32),
    ],
)


def _tc_mid_body(acc_ref, dis_ref, wa_ref, ba_ref, wb_ref, bb_ref,
                 y_ref, s_ref):
    dis = dis_ref[...]
    a1 = acc_ref[0].reshape(_R, H)
    a2 = acc_ref[1].reshape(_R, H)
    y1 = jnp.maximum(a1 * dis, 0.0)
    y2 = jnp.maximum(a2 * dis, 0.0)
    y_ref[0] = y1
    y_ref[1] = y2
    s_ref[0] = (jnp.dot(y1, wa_ref[...],
                        preferred_element_type=jnp.float32) + ba_ref[...]) * dis
    s_ref[1] = (jnp.dot(y2, wb_ref[...],
                        preferred_element_type=jnp.float32) + bb_ref[...]) * dis


_tc_mid = pl.pallas_call(
    _tc_mid_body,
    grid=(_GRID,),
    in_specs=[
        pl.BlockSpec((NC, 2 * _R, H // 2), lambda i: (0, i, 0)),
        pl.BlockSpec((_R, 1), lambda i: (i, 0)),
        pl.BlockSpec((H, H), lambda i: (0, 0)),
        pl.BlockSpec((1, H), lambda i: (0, 0)),
        pl.BlockSpec((H, H), lambda i: (0, 0)),
        pl.BlockSpec((1, H), lambda i: (0, 0)),
    ],
    out_specs=[
        pl.BlockSpec((NC, _R, H), lambda i: (0, i, 0)),
        pl.BlockSpec((NC, _R, H), lambda i: (0, i, 0)),
    ],
    out_shape=[
        jax.ShapeDtypeStruct((NC, N, H), jnp.float32),
        jax.ShapeDtypeStruct((NC, N, H), jnp.float32),
    ],
)


def _tc_fin_body(acc_ref, dis_ref, y_ref):
    dis = dis_ref[...]
    y_ref[0] = jnp.maximum(acc_ref[0].reshape(_R, H) * dis, 0.0)
    y_ref[1] = jnp.maximum(acc_ref[1].reshape(_R, H) * dis, 0.0)


_tc_fin = pl.pallas_call(
    _tc_fin_body,
    grid=(_GRID,),
    in_specs=[
        pl.BlockSpec((NC, 2 * _R, H // 2), lambda i: (0, i, 0)),
        pl.BlockSpec((_R, 1), lambda i: (i, 0)),
    ],
    out_specs=pl.BlockSpec((NC, _R, H), lambda i: (0, i, 0)),
    out_shape=jax.ShapeDtypeStruct((NC, N, H), jnp.float32),
)


def kernel(x, view2, edge_index, D_inv,
           W1, b1, W2, b2, W3, b3, W4, b4, W5, b5, W6, b6):
    del D_inv
    esrc = edge_index[0]
    edst = edge_index[1]
    esrc3 = esrc.reshape(NC, NT, WT_E)
    edst3 = edst.reshape(NC, NT, WT_E)
    edst4a = edst.reshape(NC, NT, NBLK_A, KA)
    ones_a = jnp.ones((KA,), jnp.float32)
    zeros_n = jnp.zeros((N,), jnp.float32)
    zeros_nq = jnp.zeros((ROWS_PER_TILE, H // 2), jnp.float32)
    b1r = b1.reshape(1, H)
    b2r = b2.reshape(1, H)
    b3r = b3.reshape(1, H)
    b4r = b4.reshape(1, H)
    b5r = b5.reshape(1, H)
    b6r = b6.reshape(1, H)

    deg2 = _deg_kernel(edst4a, ones_a, zeros_n)
    dega = deg2[0].reshape(N, 1)
    degb = deg2[1].reshape(N, 1)

    srcl, dstl, cnts = _bucket_kernel(esrc3, edst3)
    srcl = srcl.reshape(32, 2, LCAP_CH, IC, 2 * KB)
    dstl = dstl.reshape(32, 2, LCAP_CH, IC, 2 * KB)

    s1, dis = _tc1(x, view2, dega, degb, W1, b1r, W4, b4r)
    acc1 = _scatter_kernel(s1.reshape(NC, 2 * N, H // 2),
                           srcl, dstl, cnts, zeros_nq)
    y1, s2 = _tc_mid(acc1, dis, W2, b2r, W5, b5r)
    acc2 = _scatter_kernel(s2.reshape(NC, 2 * N, H // 2),
                           srcl, dstl, cnts, zeros_nq)
    y2, s3 = _tc_mid(acc2, dis, W3, b3r, W6, b6r)
    acc3 = _scatter_kernel(s3.reshape(NC, 2 * N, H // 2),
                           srcl, dstl, cnts, zeros_nq)
    y3 = _tc_fin(acc3, dis)

    q = jnp.concatenate([y1[0], y2[0], y3[0]], axis=1)
    p = jnp.concatenate([y1[1], y2[1], y3[1]], axis=1)
    return (q, p)


# speedup vs baseline: 7.0714x; 1.2066x over previous
import functools

import jax
import jax.numpy as jnp
from jax import lax
from jax.experimental import pallas as pl
from jax.experimental.pallas import tpu as pltpu
from jax.experimental.pallas import tpu_sc as plsc

N = 10000
E = 320000
D_IN = 128
H = 256
NC = 2
NT = 16

NHALF = N // 2
TRASH = NHALF
ACC_R = 2 * (NHALF + 8)

KB = 64
IC = 20
CHUNK = IC * KB
WT_E = E // 32
LCAP_CH = 8
LCAP = LCAP_CH * CHUNK

KA = 100
NBLK_A = WT_E // KA

ROWS_PER_TILE = 2000
FLUSH_TILES = 2 * NHALF // ROWS_PER_TILE

_MESH = plsc.VectorSubcoreMesh(
    core_axis_name="c", subcore_axis_name="s", num_cores=NC, num_subcores=NT
)


def _deg_body(edst4, ones_hbm, zeros_hbm, deg2, dst_v, ones_v, acc_sh, sem):
    c = lax.axis_index("c")
    s = lax.axis_index("s")
    pltpu.sync_copy(edst4.at[c, s], dst_v)
    pltpu.sync_copy(ones_hbm, ones_v)

    @pl.when(s == 0)
    def _():
        pltpu.sync_copy(zeros_hbm, acc_sh)

    plsc.subcore_barrier()

    def step(j, carry):
        pltpu.sync_copy(ones_v, acc_sh.at[dst_v.at[j]], add=True)
        return carry

    lax.fori_loop(0, NBLK_A, step, 0)
    plsc.subcore_barrier()

    @pl.when(s == 0)
    def _():
        pltpu.sync_copy(acc_sh, deg2.at[c])

    del sem


_deg_kernel = pl.kernel(
    _deg_body,
    out_type=jax.ShapeDtypeStruct((NC, N), jnp.float32),
    mesh=_MESH,
    scratch_types=[
        pltpu.VMEM((NBLK_A, KA), jnp.int32),
        pltpu.VMEM((KA,), jnp.float32),
        pltpu.VMEM_SHARED((N,), jnp.float32),
        pltpu.SemaphoreType.DMA,
    ],
)


def _bucket_body(esrc3, edst3, srcl, dstl, cnts,
                 src_v, dst_v, ls, ld, hs, hd, cnt_v):
    c = lax.axis_index("c")
    s = lax.axis_index("s")
    w = c * NT + s
    pltpu.sync_copy(esrc3.at[c, s], src_v)
    pltpu.sync_copy(edst3.at[c, s], dst_v)

    par = lax.rem(lax.iota(jnp.int32, 16), 2)
    zl = par
    tl = 2 * TRASH + par

    def fill2(j, carry):
        ls[pl.ds(16 * j, 16)] = zl
        hs[pl.ds(16 * j, 16)] = zl
        ld[pl.ds(16 * j, 16)] = tl
        hd[pl.ds(16 * j, 16)] = tl
        return carry

    lax.fori_loop(0, 2 * LCAP // 16, fill2, 0)

    def part(i, carry):
        nlo, nhi = carry
        sv = src_v[pl.ds(16 * i, 16)]
        dv = dst_v[pl.ds(16 * i, 16)]
        mlo = dv < NHALF
        mhi = jnp.logical_not(mlo)
        ilo = plsc.cumsum(mlo.astype(jnp.int32))
        ihi = plsc.cumsum(mhi.astype(jnp.int32))
        plo = nlo + ilo - 1
        phi = nhi + ihi - 1
        sv2 = 2 * sv
        dlo2 = 2 * dv
        dhi2 = 2 * (dv - NHALF)
        plsc.store_scatter(ls, [2 * plo], sv2, mask=mlo)
        plsc.store_scatter(ls, [2 * plo + 1], sv2 + 1, mask=mlo)
        plsc.store_scatter(ld, [2 * plo], dlo2, mask=mlo)
        plsc.store_scatter(ld, [2 * plo + 1], dlo2 + 1, mask=mlo)
        plsc.store_scatter(hs, [2 * phi], sv2, mask=mhi)
        plsc.store_scatter(hs, [2 * phi + 1], sv2 + 1, mask=mhi)
        plsc.store_scatter(hd, [2 * phi], dhi2, mask=mhi)
        plsc.store_scatter(hd, [2 * phi + 1], dhi2 + 1, mask=mhi)
        cl = jnp.max(ilo)
        return (nlo + cl, nhi + (16 - cl))

    nlo, nhi = lax.fori_loop(0, WT_E // 16, part, (0, 0))

    pltpu.sync_copy(ls, srcl.at[w, 0])
    pltpu.sync_copy(ld, dstl.at[w, 0])
    pltpu.sync_copy(hs, srcl.at[w, 1])
    pltpu.sync_copy(hd, dstl.at[w, 1])
    cnt_v[...] = jnp.full((16,), 1, jnp.int32) * nlo
    pltpu.sync_copy(cnt_v, cnts.at[w])
    del nhi


_bucket_kernel = pl.kernel(
    _bucket_body,
    compiler_params=pltpu.CompilerParams(needs_layout_passes=False),
    out_type=(
        jax.ShapeDtypeStruct((32, 2, 2 * LCAP), jnp.int32),
        jax.ShapeDtypeStruct((32, 2, 2 * LCAP), jnp.int32),
        jax.ShapeDtypeStruct((32, 16), jnp.int32),
    ),
    mesh=_MESH,
    scratch_types=[
        pltpu.VMEM((WT_E,), jnp.int32),
        pltpu.VMEM((WT_E,), jnp.int32),
        pltpu.VMEM((2 * LCAP,), jnp.int32),
        pltpu.VMEM((2 * LCAP,), jnp.int32),
        pltpu.VMEM((2 * LCAP,), jnp.int32),
        pltpu.VMEM((2 * LCAP,), jnp.int32),
        pltpu.VMEM((16,), jnp.int32),
    ],
)


def _scatter_body(s_full, srcl, dstl, cnts, zeros_hbm, accf,
                  idx_s, idx_d, rows_a, rows_b, cnt_v, acc_sh, sem_a, sem_b):
    c = lax.axis_index("c")
    s = lax.axis_index("s")
    pltpu.sync_copy(cnts, cnt_v)
    table = s_full.at[c]
    row0 = s * ROWS_PER_TILE

    for h in range(2):
        @pl.when(s < FLUSH_TILES)
        def _():
            pltpu.sync_copy(zeros_hbm, acc_sh.at[pl.ds(row0, ROWS_PER_TILE)])

        plsc.subcore_barrier()

        for li in range(2):
            w = 2 * s + li
            nlo = cnt_v[w][0]
            n = nlo if h == 0 else WT_E - nlo
            nch = (n + (CHUNK - 1)) // CHUNK

            def chunk(ch, carry):
                rem = n - ch * CHUNK
                nblk = jnp.minimum((rem + (KB - 1)) // KB, IC)
                nb2 = (nblk + 1) // 2
                pltpu.sync_copy(srcl.at[w, h, ch], idx_s)
                pltpu.sync_copy(dstl.at[w, h, ch], idx_d)
                pltpu.async_copy(table.at[idx_s.at[0]], rows_a, sem_a)

                def step(i, carry2):
                    b0 = 2 * i
                    b1 = b0 + 1
                    pltpu.make_async_copy(table.at[idx_s.at[b0]],
                                          rows_a, sem_a).wait()
                    pltpu.async_copy(table.at[idx_s.at[b1]], rows_b, sem_b)
                    pltpu.sync_copy(rows_a,
                                    acc_sh.at[idx_d.at[b0]], add=True)
                    pltpu.make_async_copy(table.at[idx_s.at[b1]],
                                          rows_b, sem_b).wait()

                    @pl.when(i < nb2 - 1)
                    def _():
                        pltpu.async_copy(table.at[idx_s.at[b0 + 2]],
                                         rows_a, sem_a)

                    pltpu.sync_copy(rows_b,
                                    acc_sh.at[idx_d.at[b1]], add=True)
                    return carry2

                lax.fori_loop(0, nb2, step, 0)
                return carry

            lax.fori_loop(0, nch, chunk, 0)

        plsc.subcore_barrier()

        @pl.when(s < FLUSH_TILES)
        def _():
            pltpu.sync_copy(
                acc_sh.at[pl.ds(row0, ROWS_PER_TILE)],
                accf.at[c, pl.ds(h * N + row0, ROWS_PER_TILE)])

        plsc.subcore_barrier()


_scatter_kernel = pl.kernel(
    _scatter_body,
    out_type=jax.ShapeDtypeStruct((NC, 2 * N, H // 2), jnp.float32),
    mesh=_MESH,
    scratch_types=[
        pltpu.VMEM((IC, 2 * KB), jnp.int32),
        pltpu.VMEM((IC, 2 * KB), jnp.int32),
        pltpu.VMEM((2 * KB, H // 2), jnp.float32),
        pltpu.VMEM((2 * KB, H // 2), jnp.float32),
        pltpu.VMEM((32, 16), jnp.int32),
        pltpu.VMEM_SHARED((ACC_R, H // 2), jnp.float32),
        pltpu.SemaphoreType.DMA,
        pltpu.SemaphoreType.DMA,
    ],
)


_R = 2000
_GRID = N // _R


def _tc1_body(x_ref, v2_ref, dega_ref, degb_ref,
              w1_ref, b1_ref, w4_ref, b4_ref, s_ref, dis_ref):
    deg = dega_ref[...] + degb_ref[...]
    dis = jnp.where(deg > 0, lax.rsqrt(jnp.maximum(deg, 1.0)), 0.0)
    dis_ref[...] = dis
    s_ref[0] = (jnp.dot(x_ref[...], w1_ref[...],
                        preferred_element_type=jnp.float32) + b1_ref[...]) * dis
    s_ref[1] = (jnp.dot(v2_ref[...], w4_ref[...],
                        preferred_element_type=jnp.float32) + b4_ref[...]) * dis


_tc1 = pl.pallas_call(
    _tc1_body,
    grid=(_GRID,),
    in_specs=[
        pl.BlockSpec((_R, D_IN), lambda i: (i, 0)),
        pl.BlockSpec((_R, D_IN), lambda i: (i, 0)),
        pl.BlockSpec((_R, 1), lambda i: (i, 0)),
        pl.BlockSpec((_R, 1), lambda i: (i, 0)),
        pl.BlockSpec((D_IN, H), lambda i: (0, 0)),
        pl.BlockSpec((1, H), lambda i: (0, 0)),
        pl.BlockSpec((D_IN, H), lambda i: (0, 0)),
        pl.BlockSpec((1, H), lambda i: (0, 0)),
    ],
    out_specs=[
        pl.BlockSpec((NC, _R, H), lambda i: (0, i, 0)),
        pl.BlockSpec((_R, 1), lambda i: (i, 0)),
    ],
    out_shape=[
        jax.ShapeDtypeStruct((NC, N, H), jnp.float32),
        jax.ShapeDtypeStruct((N, 1), jnp.float32),
    ],
)


def _tc_mid_body(acc_ref, dis_ref, wa_ref, ba_ref, wb_ref, bb_ref,
                 y_ref, s_ref):
    dis = dis_ref[...]
    a1 = acc_ref[0].reshape(_R, H)
    a2 = acc_ref[1].reshape(_R, H)
    y1 = jnp.maximum(a1 * dis, 0.0)
    y2 = jnp.maximum(a2 * dis, 0.0)
    y_ref[0] = y1
    y_ref[1] = y2
    s_ref[0] = (jnp.dot(y1, wa_ref[...],
                        preferred_element_type=jnp.float32) + ba_ref[...]) * dis
    s_ref[1] = (jnp.dot(y2, wb_ref[...],
                        preferred_element_type=jnp.float32) + bb_ref[...]) * dis


_tc_mid = pl.pallas_call(
    _tc_mid_body,
    grid=(_GRID,),
    in_specs=[
        pl.BlockSpec((NC, 2 * _R, H // 2), lambda i: (0, i, 0)),
        pl.BlockSpec((_R, 1), lambda i: (i, 0)),
        pl.BlockSpec((H, H), lambda i: (0, 0)),
        pl.BlockSpec((1, H), lambda i: (0, 0)),
        pl.BlockSpec((H, H), lambda i: (0, 0)),
        pl.BlockSpec((1, H), lambda i: (0, 0)),
    ],
    out_specs=[
        pl.BlockSpec((NC, _R, H), lambda i: (0, i, 0)),
        pl.BlockSpec((NC, _R, H), lambda i: (0, i, 0)),
    ],
    out_shape=[
        jax.ShapeDtypeStruct((NC, N, H), jnp.float32),
        jax.ShapeDtypeStruct((NC, N, H), jnp.float32),
    ],
)


def _tc_fin_body(acc_ref, dis_ref, y_ref):
    dis = dis_ref[...]
    y_ref[0] = jnp.maximum(acc_ref[0].reshape(_R, H) * dis, 0.0)
    y_ref[1] = jnp.maximum(acc_ref[1].reshape(_R, H) * dis, 0.0)


_tc_fin = pl.pallas_call(
    _tc_fin_body,
    grid=(_GRID,),
    in_specs=[
        pl.BlockSpec((NC, 2 * _R, H // 2), lambda i: (0, i, 0)),
        pl.BlockSpec((_R, 1), lambda i: (i, 0)),
    ],
    out_specs=pl.BlockSpec((NC, _R, H), lambda i: (0, i, 0)),
    out_shape=jax.ShapeDtypeStruct((NC, N, H), jnp.float32),
)


def kernel(x, view2, edge_index, D_inv,
           W1, b1, W2, b2, W3, b3, W4, b4, W5, b5, W6, b6):
    del D_inv
    esrc = edge_index[0]
    edst = edge_index[1]
    esrc3 = esrc.reshape(NC, NT, WT_E)
    edst3 = edst.reshape(NC, NT, WT_E)
    edst4a = edst.reshape(NC, NT, NBLK_A, KA)
    ones_a = jnp.ones((KA,), jnp.float32)
    zeros_n = jnp.zeros((N,), jnp.float32)
    zeros_nq = jnp.zeros((ROWS_PER_TILE, H // 2), jnp.float32)
    b1r = b1.reshape(1, H)
    b2r = b2.reshape(1, H)
    b3r = b3.reshape(1, H)
    b4r = b4.reshape(1, H)
    b5r = b5.reshape(1, H)
    b6r = b6.reshape(1, H)

    deg2 = _deg_kernel(edst4a, ones_a, zeros_n)
    dega = deg2[0].reshape(N, 1)
    degb = deg2[1].reshape(N, 1)

    srcl, dstl, cnts = _bucket_kernel(esrc3, edst3)
    srcl = srcl.reshape(32, 2, LCAP_CH, IC, 2 * KB)
    dstl = dstl.reshape(32, 2, LCAP_CH, IC, 2 * KB)

    s1, dis = _tc1(x, view2, dega, degb, W1, b1r, W4, b4r)
    acc1 = _scatter_kernel(s1.reshape(NC, 2 * N, H // 2),
                           srcl, dstl, cnts, zeros_nq)
    y1, s2 = _tc_mid(acc1, dis, W2, b2r, W5, b5r)
    acc2 = _scatter_kernel(s2.reshape(NC, 2 * N, H // 2),
                           srcl, dstl, cnts, zeros_nq)
    y2, s3 = _tc_mid(acc2, dis, W3, b3r, W6, b6r)
    acc3 = _scatter_kernel(s3.reshape(NC, 2 * N, H // 2),
                           srcl, dstl, cnts, zeros_nq)
    y3 = _tc_fin(acc3, dis)

    q = jnp.concatenate([y1[0], y2[0], y3[0]], axis=1)
    p = jnp.concatenate([y1[1], y2[1], y3[1]], axis=1)
    return (q, p)


# speedup vs baseline: 7.2207x; 1.0211x over previous
import functools

import jax
import jax.numpy as jnp
from jax import lax
from jax.experimental import pallas as pl
from jax.experimental.pallas import tpu as pltpu
from jax.experimental.pallas import tpu_sc as plsc

N = 10000
E = 320000
D_IN = 128
H = 256
NC = 2
NT = 16

NHALF = N // 2
TRASH = NHALF
ACC_R = 2 * (NHALF + 8)

KB = 64
IC = 40
CHUNK = IC * KB
WT_E = E // 32
LCAP_CH = 4
LCAP = LCAP_CH * CHUNK

KA = 100
NBLK_A = WT_E // KA

ROWS_PER_TILE = 2000
FLUSH_TILES = 2 * NHALF // ROWS_PER_TILE

_MESH = plsc.VectorSubcoreMesh(
    core_axis_name="c", subcore_axis_name="s", num_cores=NC, num_subcores=NT
)


def _deg_body(edst4, ones_hbm, zeros_hbm, deg2, dst_v, ones_v, acc_sh, sem):
    c = lax.axis_index("c")
    s = lax.axis_index("s")
    pltpu.sync_copy(edst4.at[c, s], dst_v)
    pltpu.sync_copy(ones_hbm, ones_v)

    @pl.when(s == 0)
    def _():
        pltpu.sync_copy(zeros_hbm, acc_sh)

    plsc.subcore_barrier()

    def step(j, carry):
        pltpu.sync_copy(ones_v, acc_sh.at[dst_v.at[j]], add=True)
        return carry

    lax.fori_loop(0, NBLK_A, step, 0)
    plsc.subcore_barrier()

    @pl.when(s == 0)
    def _():
        pltpu.sync_copy(acc_sh, deg2.at[c])

    del sem


_deg_kernel = pl.kernel(
    _deg_body,
    out_type=jax.ShapeDtypeStruct((NC, N), jnp.float32),
    mesh=_MESH,
    scratch_types=[
        pltpu.VMEM((NBLK_A, KA), jnp.int32),
        pltpu.VMEM((KA,), jnp.float32),
        pltpu.VMEM_SHARED((N,), jnp.float32),
        pltpu.SemaphoreType.DMA,
    ],
)


def _bucket_body(esrc3, edst3, srcl, dstl, cnts,
                 src_v, dst_v, ls, ld, hs, hd, cnt_v):
    c = lax.axis_index("c")
    s = lax.axis_index("s")
    w = c * NT + s
    pltpu.sync_copy(esrc3.at[c, s], src_v)
    pltpu.sync_copy(edst3.at[c, s], dst_v)

    par = lax.rem(lax.iota(jnp.int32, 16), 2)
    zl = par
    tl = 2 * TRASH + par

    def fill2(j, carry):
        ls[pl.ds(16 * j, 16)] = zl
        hs[pl.ds(16 * j, 16)] = zl
        ld[pl.ds(16 * j, 16)] = tl
        hd[pl.ds(16 * j, 16)] = tl
        return carry

    lax.fori_loop(0, 2 * LCAP // 16, fill2, 0)

    def part(i, carry):
        nlo, nhi = carry
        sv = src_v[pl.ds(16 * i, 16)]
        dv = dst_v[pl.ds(16 * i, 16)]
        mlo = dv < NHALF
        mhi = jnp.logical_not(mlo)
        ilo = plsc.cumsum(mlo.astype(jnp.int32))
        ihi = plsc.cumsum(mhi.astype(jnp.int32))
        plo = nlo + ilo - 1
        phi = nhi + ihi - 1
        sv2 = 2 * sv
        dlo2 = 2 * dv
        dhi2 = 2 * (dv - NHALF)
        plsc.store_scatter(ls, [2 * plo], sv2, mask=mlo)
        plsc.store_scatter(ls, [2 * plo + 1], sv2 + 1, mask=mlo)
        plsc.store_scatter(ld, [2 * plo], dlo2, mask=mlo)
        plsc.store_scatter(ld, [2 * plo + 1], dlo2 + 1, mask=mlo)
        plsc.store_scatter(hs, [2 * phi], sv2, mask=mhi)
        plsc.store_scatter(hs, [2 * phi + 1], sv2 + 1, mask=mhi)
        plsc.store_scatter(hd, [2 * phi], dhi2, mask=mhi)
        plsc.store_scatter(hd, [2 * phi + 1], dhi2 + 1, mask=mhi)
        cl = jnp.max(ilo)
        return (nlo + cl, nhi + (16 - cl))

    nlo, nhi = lax.fori_loop(0, WT_E // 16, part, (0, 0))

    pltpu.sync_copy(ls, srcl.at[w, 0])
    pltpu.sync_copy(ld, dstl.at[w, 0])
    pltpu.sync_copy(hs, srcl.at[w, 1])
    pltpu.sync_copy(hd, dstl.at[w, 1])
    cnt_v[...] = jnp.full((16,), 1, jnp.int32) * nlo
    pltpu.sync_copy(cnt_v, cnts.at[w])
    del nhi


_bucket_kernel = pl.kernel(
    _bucket_body,
    compiler_params=pltpu.CompilerParams(needs_layout_passes=False),
    out_type=(
        jax.ShapeDtypeStruct((32, 2, 2 * LCAP), jnp.int32),
        jax.ShapeDtypeStruct((32, 2, 2 * LCAP), jnp.int32),
        jax.ShapeDtypeStruct((32, 16), jnp.int32),
    ),
    mesh=_MESH,
    scratch_types=[
        pltpu.VMEM((WT_E,), jnp.int32),
        pltpu.VMEM((WT_E,), jnp.int32),
        pltpu.VMEM((2 * LCAP,), jnp.int32),
        pltpu.VMEM((2 * LCAP,), jnp.int32),
        pltpu.VMEM((2 * LCAP,), jnp.int32),
        pltpu.VMEM((2 * LCAP,), jnp.int32),
        pltpu.VMEM((16,), jnp.int32),
    ],
)


def _scatter_body(s_full, srcl, dstl, cnts, zeros_hbm, accf,
                  idx_s, idx_d, rows_a, rows_b, cnt_v, acc_sh, sem_a, sem_b):
    c = lax.axis_index("c")
    s = lax.axis_index("s")
    pltpu.sync_copy(cnts, cnt_v)
    table = s_full.at[c]
    row0 = s * ROWS_PER_TILE

    for h in range(2):
        @pl.when(s < FLUSH_TILES)
        def _():
            pltpu.sync_copy(zeros_hbm, acc_sh.at[pl.ds(row0, ROWS_PER_TILE)])

        plsc.subcore_barrier()

        for li in range(2):
            w = 2 * s + li
            nlo = cnt_v[w][0]
            n = nlo if h == 0 else WT_E - nlo
            nch = (n + (CHUNK - 1)) // CHUNK

            def chunk(ch, carry):
                rem = n - ch * CHUNK
                nblk = jnp.minimum((rem + (KB - 1)) // KB, IC)
                nb2 = (nblk + 1) // 2
                pltpu.sync_copy(srcl.at[w, h, ch], idx_s)
                pltpu.sync_copy(dstl.at[w, h, ch], idx_d)
                pltpu.async_copy(table.at[idx_s.at[0]], rows_a, sem_a)

                def step(i, carry2):
                    b0 = 2 * i
                    b1 = b0 + 1
                    pltpu.make_async_copy(table.at[idx_s.at[b0]],
                                          rows_a, sem_a).wait()
                    pltpu.async_copy(table.at[idx_s.at[b1]], rows_b, sem_b)
                    pltpu.sync_copy(rows_a,
                                    acc_sh.at[idx_d.at[b0]], add=True)
                    pltpu.make_async_copy(table.at[idx_s.at[b1]],
                                          rows_b, sem_b).wait()

                    @pl.when(i < nb2 - 1)
                    def _():
                        pltpu.async_copy(table.at[idx_s.at[b0 + 2]],
                                         rows_a, sem_a)

                    pltpu.sync_copy(rows_b,
                                    acc_sh.at[idx_d.at[b1]], add=True)
                    return carry2

                lax.fori_loop(0, nb2, step, 0)
                return carry

            lax.fori_loop(0, nch, chunk, 0)

        plsc.subcore_barrier()

        @pl.when(s < FLUSH_TILES)
        def _():
            pltpu.sync_copy(
                acc_sh.at[pl.ds(row0, ROWS_PER_TILE)],
                accf.at[c, pl.ds(h * N + row0, ROWS_PER_TILE)])

        plsc.subcore_barrier()


_scatter_kernel = pl.kernel(
    _scatter_body,
    out_type=jax.ShapeDtypeStruct((NC, 2 * N, H // 2), jnp.float32),
    mesh=_MESH,
    scratch_types=[
        pltpu.VMEM((IC, 2 * KB), jnp.int32),
        pltpu.VMEM((IC, 2 * KB), jnp.int32),
        pltpu.VMEM((2 * KB, H // 2), jnp.float32),
        pltpu.VMEM((2 * KB, H // 2), jnp.float32),
        pltpu.VMEM((32, 16), jnp.int32),
        pltpu.VMEM_SHARED((ACC_R, H // 2), jnp.float32),
        pltpu.SemaphoreType.DMA,
        pltpu.SemaphoreType.DMA,
    ],
)


_R = 2000
_GRID = N // _R


def _tc1_body(x_ref, v2_ref, dega_ref, degb_ref,
              w1_ref, b1_ref, w4_ref, b4_ref, s_ref, dis_ref):
    deg = dega_ref[...] + degb_ref[...]
    dis = jnp.where(deg > 0, lax.rsqrt(jnp.maximum(deg, 1.0)), 0.0)
    dis_ref[...] = dis
    s_ref[0] = (jnp.dot(x_ref[...], w1_ref[...],
                        preferred_element_type=jnp.float32) + b1_ref[...]) * dis
    s_ref[1] = (jnp.dot(v2_ref[...], w4_ref[...],
                        preferred_element_type=jnp.float32) + b4_ref[...]) * dis


_tc1 = pl.pallas_call(
    _tc1_body,
    grid=(_GRID,),
    in_specs=[
        pl.BlockSpec((_R, D_IN), lambda i: (i, 0)),
        pl.BlockSpec((_R, D_IN), lambda i: (i, 0)),
        pl.BlockSpec((_R, 1), lambda i: (i, 0)),
        pl.BlockSpec((_R, 1), lambda i: (i, 0)),
        pl.BlockSpec((D_IN, H), lambda i: (0, 0)),
        pl.BlockSpec((1, H), lambda i: (0, 0)),
        pl.BlockSpec((D_IN, H), lambda i: (0, 0)),
        pl.BlockSpec((1, H), lambda i: (0, 0)),
    ],
    out_specs=[
        pl.BlockSpec((NC, _R, H), lambda i: (0, i, 0)),
        pl.BlockSpec((_R, 1), lambda i: (i, 0)),
    ],
    out_shape=[
        jax.ShapeDtypeStruct((NC, N, H), jnp.float32),
        jax.ShapeDtypeStruct((N, 1), jnp.float32),
    ],
)


def _tc_mid_body(acc_ref, dis_ref, wa_ref, ba_ref, wb_ref, bb_ref,
                 y_ref, s_ref):
    dis = dis_ref[...]
    a1 = acc_ref[0].reshape(_R, H)
    a2 = acc_ref[1].reshape(_R, H)
    y1 = jnp.maximum(a1 * dis, 0.0)
    y2 = jnp.maximum(a2 * dis, 0.0)
    y_ref[0] = y1
    y_ref[1] = y2
    s_ref[0] = (jnp.dot(y1, wa_ref[...],
                        preferred_element_type=jnp.float32) + ba_ref[...]) * dis
    s_ref[1] = (jnp.dot(y2, wb_ref[...],
                        preferred_element_type=jnp.float32) + bb_ref[...]) * dis


_tc_mid = pl.pallas_call(
    _tc_mid_body,
    grid=(_GRID,),
    in_specs=[
        pl.BlockSpec((NC, 2 * _R, H // 2), lambda i: (0, i, 0)),
        pl.BlockSpec((_R, 1), lambda i: (i, 0)),
        pl.BlockSpec((H, H), lambda i: (0, 0)),
        pl.BlockSpec((1, H), lambda i: (0, 0)),
        pl.BlockSpec((H, H), lambda i: (0, 0)),
        pl.BlockSpec((1, H), lambda i: (0, 0)),
    ],
    out_specs=[
        pl.BlockSpec((NC, _R, H), lambda i: (0, i, 0)),
        pl.BlockSpec((NC, _R, H), lambda i: (0, i, 0)),
    ],
    out_shape=[
        jax.ShapeDtypeStruct((NC, N, H), jnp.float32),
        jax.ShapeDtypeStruct((NC, N, H), jnp.float32),
    ],
)


def _tc_fin_body(acc_ref, dis_ref, y_ref):
    dis = dis_ref[...]
    y_ref[0] = jnp.maximum(acc_ref[0].reshape(_R, H) * dis, 0.0)
    y_ref[1] = jnp.maximum(acc_ref[1].reshape(_R, H) * dis, 0.0)


_tc_fin = pl.pallas_call(
    _tc_fin_body,
    grid=(_GRID,),
    in_specs=[
        pl.BlockSpec((NC, 2 * _R, H // 2), lambda i: (0, i, 0)),
        pl.BlockSpec((_R, 1), lambda i: (i, 0)),
    ],
    out_specs=pl.BlockSpec((NC, _R, H), lambda i: (0, i, 0)),
    out_shape=jax.ShapeDtypeStruct((NC, N, H), jnp.float32),
)


def kernel(x, view2, edge_index, D_inv,
           W1, b1, W2, b2, W3, b3, W4, b4, W5, b5, W6, b6):
    del D_inv
    esrc = edge_index[0]
    edst = edge_index[1]
    esrc3 = esrc.reshape(NC, NT, WT_E)
    edst3 = edst.reshape(NC, NT, WT_E)
    edst4a = edst.reshape(NC, NT, NBLK_A, KA)
    ones_a = jnp.ones((KA,), jnp.float32)
    zeros_n = jnp.zeros((N,), jnp.float32)
    zeros_nq = jnp.zeros((ROWS_PER_TILE, H // 2), jnp.float32)
    b1r = b1.reshape(1, H)
    b2r = b2.reshape(1, H)
    b3r = b3.reshape(1, H)
    b4r = b4.reshape(1, H)
    b5r = b5.reshape(1, H)
    b6r = b6.reshape(1, H)

    deg2 = _deg_kernel(edst4a, ones_a, zeros_n)
    dega = deg2[0].reshape(N, 1)
    degb = deg2[1].reshape(N, 1)

    srcl, dstl, cnts = _bucket_kernel(esrc3, edst3)
    srcl = srcl.reshape(32, 2, LCAP_CH, IC, 2 * KB)
    dstl = dstl.reshape(32, 2, LCAP_CH, IC, 2 * KB)

    s1, dis = _tc1(x, view2, dega, degb, W1, b1r, W4, b4r)
    acc1 = _scatter_kernel(s1.reshape(NC, 2 * N, H // 2),
                           srcl, dstl, cnts, zeros_nq)
    y1, s2 = _tc_mid(acc1, dis, W2, b2r, W5, b5r)
    acc2 = _scatter_kernel(s2.reshape(NC, 2 * N, H // 2),
                           srcl, dstl, cnts, zeros_nq)
    y2, s3 = _tc_mid(acc2, dis, W3, b3r, W6, b6r)
    acc3 = _scatter_kernel(s3.reshape(NC, 2 * N, H // 2),
                           srcl, dstl, cnts, zeros_nq)
    y3 = _tc_fin(acc3, dis)

    q = jnp.concatenate([y1[0], y2[0], y3[0]], axis=1)
    p = jnp.concatenate([y1[1], y2[1], y3[1]], axis=1)
    return (q, p)


# speedup vs baseline: 7.2219x; 1.0002x over previous
import jax
import jax.numpy as jnp
from jax import lax
from jax.experimental import pallas as pl
from jax.experimental.pallas import tpu as pltpu
from jax.experimental.pallas import tpu_sc as plsc

N = 10000
E = 320000
D_IN = 128
H = 256
NC = 2
NT = 16

NHALF = N // 2
TRASH = NHALF
ACC_R = 2 * (NHALF + 8)

KB = 64
IC = 40
CHUNK = IC * KB
WT_E = E // 32
LCAP_CH = 4
LCAP = LCAP_CH * CHUNK

KA = 100
NBLK_A = WT_E // KA

ROWS_PER_TILE = 2000
FLUSH_TILES = 2 * NHALF // ROWS_PER_TILE

_MESH = plsc.VectorSubcoreMesh(
    core_axis_name="c", subcore_axis_name="s", num_cores=NC, num_subcores=NT
)


def _deg_body(edst4, ones_hbm, zeros_hbm, deg2, dst_v, ones_v, acc_sh, sem):
    c = lax.axis_index("c")
    s = lax.axis_index("s")
    pltpu.sync_copy(edst4.at[c, s], dst_v)
    pltpu.sync_copy(ones_hbm, ones_v)

    @pl.when(s == 0)
    def _():
        pltpu.sync_copy(zeros_hbm, acc_sh)

    plsc.subcore_barrier()

    def step(j, carry):
        pltpu.sync_copy(ones_v, acc_sh.at[dst_v.at[j]], add=True)
        return carry

    lax.fori_loop(0, NBLK_A, step, 0)
    plsc.subcore_barrier()

    @pl.when(s == 0)
    def _():
        pltpu.sync_copy(acc_sh, deg2.at[c])

    del sem


_deg_kernel = pl.kernel(
    _deg_body,
    out_type=jax.ShapeDtypeStruct((NC, N), jnp.float32),
    mesh=_MESH,
    scratch_types=[
        pltpu.VMEM((NBLK_A, KA), jnp.int32),
        pltpu.VMEM((KA,), jnp.float32),
        pltpu.VMEM_SHARED((N,), jnp.float32),
        pltpu.SemaphoreType.DMA,
    ],
)


def _bucket_body(esrc3, edst3, srcl, dstl, cnts,
                 src_v, dst_v, ls, ld, hs, hd, cnt_v):
    c = lax.axis_index("c")
    s = lax.axis_index("s")
    w = c * NT + s
    pltpu.sync_copy(esrc3.at[c, s], src_v)
    pltpu.sync_copy(edst3.at[c, s], dst_v)

    par = lax.rem(lax.iota(jnp.int32, 16), 2)
    zl = par
    tl = 2 * TRASH + par

    def fill2(j, carry):
        ls[pl.ds(16 * j, 16)] = zl
        hs[pl.ds(16 * j, 16)] = zl
        ld[pl.ds(16 * j, 16)] = tl
        hd[pl.ds(16 * j, 16)] = tl
        return carry

    lax.fori_loop(0, 2 * LCAP // 16, fill2, 0)

    def part(i, carry):
        nlo, nhi = carry
        sv = src_v[pl.ds(16 * i, 16)]
        dv = dst_v[pl.ds(16 * i, 16)]
        mlo = dv < NHALF
        mhi = jnp.logical_not(mlo)
        ilo = plsc.cumsum(mlo.astype(jnp.int32))
        ihi = plsc.cumsum(mhi.astype(jnp.int32))
        plo = nlo + ilo - 1
        phi = nhi + ihi - 1
        sv2 = 2 * sv
        dlo2 = 2 * dv
        dhi2 = 2 * (dv - NHALF)
        plsc.store_scatter(ls, [2 * plo], sv2, mask=mlo)
        plsc.store_scatter(ls, [2 * plo + 1], sv2 + 1, mask=mlo)
        plsc.store_scatter(ld, [2 * plo], dlo2, mask=mlo)
        plsc.store_scatter(ld, [2 * plo + 1], dlo2 + 1, mask=mlo)
        plsc.store_scatter(hs, [2 * phi], sv2, mask=mhi)
        plsc.store_scatter(hs, [2 * phi + 1], sv2 + 1, mask=mhi)
        plsc.store_scatter(hd, [2 * phi], dhi2, mask=mhi)
        plsc.store_scatter(hd, [2 * phi + 1], dhi2 + 1, mask=mhi)
        cl = jnp.max(ilo)
        return (nlo + cl, nhi + (16 - cl))

    nlo, nhi = lax.fori_loop(0, WT_E // 16, part, (0, 0))

    pltpu.sync_copy(ls, srcl.at[w, 0])
    pltpu.sync_copy(ld, dstl.at[w, 0])
    pltpu.sync_copy(hs, srcl.at[w, 1])
    pltpu.sync_copy(hd, dstl.at[w, 1])
    cnt_v[...] = jnp.full((16,), 1, jnp.int32) * nlo
    pltpu.sync_copy(cnt_v, cnts.at[w])
    del nhi


_bucket_kernel = pl.kernel(
    _bucket_body,
    compiler_params=pltpu.CompilerParams(needs_layout_passes=False),
    out_type=(
        jax.ShapeDtypeStruct((32, 2, 2 * LCAP), jnp.int32),
        jax.ShapeDtypeStruct((32, 2, 2 * LCAP), jnp.int32),
        jax.ShapeDtypeStruct((32, 16), jnp.int32),
    ),
    mesh=_MESH,
    scratch_types=[
        pltpu.VMEM((WT_E,), jnp.int32),
        pltpu.VMEM((WT_E,), jnp.int32),
        pltpu.VMEM((2 * LCAP,), jnp.int32),
        pltpu.VMEM((2 * LCAP,), jnp.int32),
        pltpu.VMEM((2 * LCAP,), jnp.int32),
        pltpu.VMEM((2 * LCAP,), jnp.int32),
        pltpu.VMEM((16,), jnp.int32),
    ],
)


def _scatter_body(s_full, srcl, dstl, cnts, zeros_hbm, accf,
                  idx_s, idx_d, rows_a, rows_b, cnt_v, acc_sh, sem_a, sem_b):
    c = lax.axis_index("c")
    s = lax.axis_index("s")
    pltpu.sync_copy(cnts, cnt_v)
    table = s_full.at[c]
    row0 = s * ROWS_PER_TILE

    for h in range(2):
        @pl.when(s < FLUSH_TILES)
        def _():
            pltpu.sync_copy(zeros_hbm, acc_sh.at[pl.ds(row0, ROWS_PER_TILE)])

        plsc.subcore_barrier()

        for li in range(2):
            w = 2 * s + li
            nlo = cnt_v[w][0]
            n = nlo if h == 0 else WT_E - nlo
            nch = (n + (CHUNK - 1)) // CHUNK

            def chunk(ch, carry):
                rem = n - ch * CHUNK
                nblk = jnp.minimum((rem + (KB - 1)) // KB, IC)
                nb2 = (nblk + 1) // 2
                pltpu.sync_copy(srcl.at[w, h, ch], idx_s)
                pltpu.sync_copy(dstl.at[w, h, ch], idx_d)
                pltpu.async_copy(table.at[idx_s.at[0]], rows_a, sem_a)

                def step(i, carry2):
                    b0 = 2 * i
                    b1 = b0 + 1
                    pltpu.make_async_copy(table.at[idx_s.at[b0]],
                                          rows_a, sem_a).wait()
                    pltpu.async_copy(table.at[idx_s.at[b1]], rows_b, sem_b)
                    pltpu.sync_copy(rows_a,
                                    acc_sh.at[idx_d.at[b0]], add=True)
                    pltpu.make_async_copy(table.at[idx_s.at[b1]],
                                          rows_b, sem_b).wait()

                    @pl.when(i < nb2 - 1)
                    def _():
                        pltpu.async_copy(table.at[idx_s.at[b0 + 2]],
                                         rows_a, sem_a)

                    pltpu.sync_copy(rows_b,
                                    acc_sh.at[idx_d.at[b1]], add=True)
                    return carry2

                lax.fori_loop(0, nb2, step, 0)
                return carry

            lax.fori_loop(0, nch, chunk, 0)

        plsc.subcore_barrier()

        @pl.when(s < FLUSH_TILES)
        def _():
            pltpu.sync_copy(
                acc_sh.at[pl.ds(row0, ROWS_PER_TILE)],
                accf.at[c, pl.ds(h * N + row0, ROWS_PER_TILE)])

        plsc.subcore_barrier()


_scatter_kernel = pl.kernel(
    _scatter_body,
    out_type=jax.ShapeDtypeStruct((NC, 2 * N, H // 2), jnp.float32),
    mesh=_MESH,
    scratch_types=[
        pltpu.VMEM((IC, 2 * KB), jnp.int32),
        pltpu.VMEM((IC, 2 * KB), jnp.int32),
        pltpu.VMEM((2 * KB, H // 2), jnp.float32),
        pltpu.VMEM((2 * KB, H // 2), jnp.float32),
        pltpu.VMEM((32, 16), jnp.int32),
        pltpu.VMEM_SHARED((ACC_R, H // 2), jnp.float32),
        pltpu.SemaphoreType.DMA,
        pltpu.SemaphoreType.DMA,
    ],
)


_R = 2000
_GRID = N // _R


def _tc1_body(x_ref, v2_ref, dega_ref, degb_ref,
              w1_ref, b1_ref, w4_ref, b4_ref, s_ref, dis_ref):
    deg = dega_ref[...] + degb_ref[...]
    dis = jnp.where(deg > 0, lax.rsqrt(jnp.maximum(deg, 1.0)), 0.0)
    dis_ref[...] = dis
    s_ref[0] = (jnp.dot(x_ref[...], w1_ref[...],
                        preferred_element_type=jnp.float32) + b1_ref[...]) * dis
    s_ref[1] = (jnp.dot(v2_ref[...], w4_ref[...],
                        preferred_element_type=jnp.float32) + b4_ref[...]) * dis


_tc1 = pl.pallas_call(
    _tc1_body,
    grid=(_GRID,),
    in_specs=[
        pl.BlockSpec((_R, D_IN), lambda i: (i, 0)),
        pl.BlockSpec((_R, D_IN), lambda i: (i, 0)),
        pl.BlockSpec((_R, 1), lambda i: (i, 0)),
        pl.BlockSpec((_R, 1), lambda i: (i, 0)),
        pl.BlockSpec((D_IN, H), lambda i: (0, 0)),
        pl.BlockSpec((1, H), lambda i: (0, 0)),
        pl.BlockSpec((D_IN, H), lambda i: (0, 0)),
        pl.BlockSpec((1, H), lambda i: (0, 0)),
    ],
    out_specs=[
        pl.BlockSpec((NC, _R, H), lambda i: (0, i, 0)),
        pl.BlockSpec((_R, 1), lambda i: (i, 0)),
    ],
    out_shape=[
        jax.ShapeDtypeStruct((NC, N, H), jnp.float32),
        jax.ShapeDtypeStruct((N, 1), jnp.float32),
    ],
)


def _tc_mid_body(acc_ref, dis_ref, wa_ref, ba_ref, wb_ref, bb_ref,
                 y_ref, s_ref):
    dis = dis_ref[...]
    a1 = acc_ref[0].reshape(_R, H)
    a2 = acc_ref[1].reshape(_R, H)
    y1 = jnp.maximum(a1 * dis, 0.0)
    y2 = jnp.maximum(a2 * dis, 0.0)
    y_ref[0] = y1
    y_ref[1] = y2
    s_ref[0] = (jnp.dot(y1, wa_ref[...],
                        preferred_element_type=jnp.float32) + ba_ref[...]) * dis
    s_ref[1] = (jnp.dot(y2, wb_ref[...],
                        preferred_element_type=jnp.float32) + bb_ref[...]) * dis


_tc_mid = pl.pallas_call(
    _tc_mid_body,
    grid=(_GRID,),
    in_specs=[
        pl.BlockSpec((NC, 2 * _R, H // 2), lambda i: (0, i, 0)),
        pl.BlockSpec((_R, 1), lambda i: (i, 0)),
        pl.BlockSpec((H, H), lambda i: (0, 0)),
        pl.BlockSpec((1, H), lambda i: (0, 0)),
        pl.BlockSpec((H, H), lambda i: (0, 0)),
        pl.BlockSpec((1, H), lambda i: (0, 0)),
    ],
    out_specs=[
        pl.BlockSpec((NC, _R, H), lambda i: (0, i, 0)),
        pl.BlockSpec((NC, _R, H), lambda i: (0, i, 0)),
    ],
    out_shape=[
        jax.ShapeDtypeStruct((NC, N, H), jnp.float32),
        jax.ShapeDtypeStruct((NC, N, H), jnp.float32),
    ],
)


def _tc_fin_body(acc_ref, dis_ref, y_ref):
    dis = dis_ref[...]
    y_ref[0] = jnp.maximum(acc_ref[0].reshape(_R, H) * dis, 0.0)
    y_ref[1] = jnp.maximum(acc_ref[1].reshape(_R, H) * dis, 0.0)


_tc_fin = pl.pallas_call(
    _tc_fin_body,
    grid=(_GRID,),
    in_specs=[
        pl.BlockSpec((NC, 2 * _R, H // 2), lambda i: (0, i, 0)),
        pl.BlockSpec((_R, 1), lambda i: (i, 0)),
    ],
    out_specs=pl.BlockSpec((NC, _R, H), lambda i: (0, i, 0)),
    out_shape=jax.ShapeDtypeStruct((NC, N, H), jnp.float32),
)


def kernel(x, view2, edge_index, D_inv,
           W1, b1, W2, b2, W3, b3, W4, b4, W5, b5, W6, b6):
    del D_inv
    esrc = edge_index[0]
    edst = edge_index[1]
    esrc3 = esrc.reshape(NC, NT, WT_E)
    edst3 = edst.reshape(NC, NT, WT_E)
    edst4a = edst.reshape(NC, NT, NBLK_A, KA)
    ones_a = jnp.ones((KA,), jnp.float32)
    zeros_n = jnp.zeros((N,), jnp.float32)
    zeros_nq = jnp.zeros((ROWS_PER_TILE, H // 2), jnp.float32)
    b1r = b1.reshape(1, H)
    b2r = b2.reshape(1, H)
    b3r = b3.reshape(1, H)
    b4r = b4.reshape(1, H)
    b5r = b5.reshape(1, H)
    b6r = b6.reshape(1, H)

    deg2 = _deg_kernel(edst4a, ones_a, zeros_n)
    dega = deg2[0].reshape(N, 1)
    degb = deg2[1].reshape(N, 1)

    srcl, dstl, cnts = _bucket_kernel(esrc3, edst3)
    srcl = srcl.reshape(32, 2, LCAP_CH, IC, 2 * KB)
    dstl = dstl.reshape(32, 2, LCAP_CH, IC, 2 * KB)

    s1, dis = _tc1(x, view2, dega, degb, W1, b1r, W4, b4r)
    acc1 = _scatter_kernel(s1.reshape(NC, 2 * N, H // 2),
                           srcl, dstl, cnts, zeros_nq)
    y1, s2 = _tc_mid(acc1, dis, W2, b2r, W5, b5r)
    acc2 = _scatter_kernel(s2.reshape(NC, 2 * N, H // 2),
                           srcl, dstl, cnts, zeros_nq)
    y2, s3 = _tc_mid(acc2, dis, W3, b3r, W6, b6r)
    acc3 = _scatter_kernel(s3.reshape(NC, 2 * N, H // 2),
                           srcl, dstl, cnts, zeros_nq)
    y3 = _tc_fin(acc3, dis)

    q = jnp.concatenate([y1[0], y2[0], y3[0]], axis=1)
    p = jnp.concatenate([y1[1], y2[1], y3[1]], axis=1)
    return (q, p)
